# baseline (speedup 1.0000x reference)
import functools

import jax
import jax.numpy as jnp
from jax import lax
from jax.experimental import pallas as pl
from jax.experimental.pallas import tpu as pltpu
from jax.experimental.pallas import tpu_sc as plsc

N = 1_000_000
NPAIR = 262_144

NT = 16
SEG = 62_528
SEG_LAST = N - (NT - 1) * SEG
NSTAGE = 4
STG = SEG // NSTAGE
STG_LAST = SEG_LAST // NSTAGE

CHUNK = 128
PAIRS_PER_TILE = NPAIR // NT
NCHUNK = PAIRS_PER_TILE // CHUNK


def _sc_scatter_add(inp, idx2, val2):
    mesh = plsc.VectorSubcoreMesh(core_axis_name="c", subcore_axis_name="s",
                                  num_cores=1)

    @functools.partial(
        pl.kernel,
        mesh=mesh,
        out_type=jax.ShapeDtypeStruct((N,), jnp.int32),
        scratch_types=[
            pltpu.VMEM_SHARED((N,), jnp.int32),
            pltpu.VMEM((NCHUNK, CHUNK), jnp.int32),
            pltpu.VMEM((NCHUNK, CHUNK), jnp.int32),
            pltpu.VMEM((STG,), jnp.int32),
        ],
    )
    def k(in_hbm, idx_hbm, val_hbm, out_hbm, acc_sh, idx_v, val_v, stage_v):
        tid = lax.axis_index("s")

        row0 = tid * NCHUNK
        pltpu.sync_copy(idx_hbm.at[pl.ds(row0, NCHUNK)], idx_v)
        pltpu.sync_copy(val_hbm.at[pl.ds(row0, NCHUNK)], val_v)

        @pl.when(tid < NT - 1)
        def _():
            for c in range(NSTAGE):
                off = tid * SEG + c * STG
                pltpu.sync_copy(in_hbm.at[pl.ds(off, STG)], stage_v)
                pltpu.sync_copy(stage_v, acc_sh.at[pl.ds(off, STG)])

        @pl.when(tid == NT - 1)
        def _():
            for c in range(NSTAGE):
                off = tid * SEG + c * STG_LAST
                pltpu.sync_copy(in_hbm.at[pl.ds(off, STG_LAST)],
                                stage_v.at[pl.ds(0, STG_LAST)])
                pltpu.sync_copy(stage_v.at[pl.ds(0, STG_LAST)],
                                acc_sh.at[pl.ds(off, STG_LAST)])

        plsc.subcore_barrier()

        def body(j, carry):
            pltpu.sync_copy(val_v.at[j], acc_sh.at[idx_v.at[j]], add=True)
            return carry

        lax.fori_loop(0, NCHUNK, body, 0)

        plsc.subcore_barrier()

        @pl.when(tid < NT - 1)
        def _():
            for c in range(NSTAGE):
                off = tid * SEG + c * STG
                pltpu.sync_copy(acc_sh.at[pl.ds(off, STG)], stage_v)
                pltpu.sync_copy(stage_v, out_hbm.at[pl.ds(off, STG)])

        @pl.when(tid == NT - 1)
        def _():
            for c in range(NSTAGE):
                off = tid * SEG + c * STG_LAST
                pltpu.sync_copy(acc_sh.at[pl.ds(off, STG_LAST)],
                                stage_v.at[pl.ds(0, STG_LAST)])
                pltpu.sync_copy(stage_v.at[pl.ds(0, STG_LAST)],
                                out_hbm.at[pl.ds(off, STG_LAST)])

    return k(inp, idx2, val2)


def kernel(input, index, value):
    idx2 = index.reshape(NPAIR // CHUNK, CHUNK)
    val2 = value.reshape(NPAIR // CHUNK, CHUNK)
    return _sc_scatter_add(input, idx2, val2)


# speedup vs baseline: 1.3319x; 1.3319x over previous
import functools

import jax
import jax.numpy as jnp
from jax import lax
from jax.experimental import pallas as pl
from jax.experimental.pallas import tpu as pltpu
from jax.experimental.pallas import tpu_sc as plsc

N = 1_000_000
NPAIR = 262_144

NT = 16
SEG = 62_528
SEG_LAST = N - (NT - 1) * SEG
NSTAGE = 2
STG = SEG // NSTAGE
STG_LAST = SEG_LAST // NSTAGE

PAIRS_PER_TILE = NPAIR // NT


def _sc_scatter_add(inp, idx, val):
    mesh = plsc.VectorSubcoreMesh(core_axis_name="c", subcore_axis_name="s",
                                  num_cores=1)

    @functools.partial(
        pl.kernel,
        mesh=mesh,
        out_type=jax.ShapeDtypeStruct((N,), jnp.int32),
        scratch_types=[
            pltpu.VMEM_SHARED((N,), jnp.int32),
            pltpu.VMEM((PAIRS_PER_TILE,), jnp.int32),
            pltpu.VMEM((PAIRS_PER_TILE,), jnp.int32),
            pltpu.VMEM((STG,), jnp.int32),
            pltpu.SemaphoreType.DMA,
        ],
    )
    def k(in_hbm, idx_hbm, val_hbm, out_hbm, acc_sh, idx_v, val_v, stage_v,
          sem):
        tid = lax.axis_index("s")

        pair0 = tid * PAIRS_PER_TILE
        cp_idx = pltpu.async_copy(idx_hbm.at[pl.ds(pair0, PAIRS_PER_TILE)],
                                  idx_v, sem)
        cp_val = pltpu.async_copy(val_hbm.at[pl.ds(pair0, PAIRS_PER_TILE)],
                                  val_v, sem)

        @pl.when(tid < NT - 1)
        def _():
            for c in range(NSTAGE):
                off = tid * SEG + c * STG
                pltpu.sync_copy(in_hbm.at[pl.ds(off, STG)], stage_v)
                pltpu.sync_copy(stage_v, acc_sh.at[pl.ds(off, STG)])

        @pl.when(tid == NT - 1)
        def _():
            for c in range(NSTAGE):
                off = tid * SEG + c * STG_LAST
                pltpu.sync_copy(in_hbm.at[pl.ds(off, STG_LAST)],
                                stage_v.at[pl.ds(0, STG_LAST)])
                pltpu.sync_copy(stage_v.at[pl.ds(0, STG_LAST)],
                                acc_sh.at[pl.ds(off, STG_LAST)])

        cp_idx.wait()
        cp_val.wait()
        plsc.subcore_barrier()

        pltpu.sync_copy(val_v, acc_sh.at[idx_v], add=True)

        plsc.subcore_barrier()

        @pl.when(tid < NT - 1)
        def _():
            for c in range(NSTAGE):
                off = tid * SEG + c * STG
                pltpu.sync_copy(acc_sh.at[pl.ds(off, STG)], stage_v)
                pltpu.sync_copy(stage_v, out_hbm.at[pl.ds(off, STG)])

        @pl.when(tid == NT - 1)
        def _():
            for c in range(NSTAGE):
                off = tid * SEG + c * STG_LAST
                pltpu.sync_copy(acc_sh.at[pl.ds(off, STG_LAST)],
                                stage_v.at[pl.ds(0, STG_LAST)])
                pltpu.sync_copy(stage_v.at[pl.ds(0, STG_LAST)],
                                out_hbm.at[pl.ds(off, STG_LAST)])

    return k(inp, idx, val)


def kernel(input, index, value):
    return _sc_scatter_add(input, index, value)


# speedup vs baseline: 1.3899x; 1.0436x over previous
import functools

import jax
import jax.numpy as jnp
from jax import lax
from jax.experimental import pallas as pl
from jax.experimental.pallas import tpu as pltpu
from jax.experimental.pallas import tpu_sc as plsc

N = 1_000_000
NPAIR = 262_144

NT = 16
SEG = 62_528
SEG_LAST = N - (NT - 1) * SEG
NSTAGE = 4
STG = SEG // NSTAGE
STG_LAST = SEG_LAST // NSTAGE

PAIRS_PER_TILE = NPAIR // NT


def _sc_scatter_add(inp, idx, val):
    mesh = plsc.VectorSubcoreMesh(core_axis_name="c", subcore_axis_name="s",
                                  num_cores=1)

    @functools.partial(
        pl.kernel,
        mesh=mesh,
        out_type=jax.ShapeDtypeStruct((N,), jnp.int32),
        scratch_types=[
            pltpu.VMEM_SHARED((N,), jnp.int32),
            pltpu.VMEM((PAIRS_PER_TILE,), jnp.int32),
            pltpu.VMEM((PAIRS_PER_TILE,), jnp.int32),
            pltpu.VMEM((STG,), jnp.int32),
            pltpu.VMEM((STG,), jnp.int32),
            pltpu.SemaphoreType.DMA,
            pltpu.SemaphoreType.DMA,
            pltpu.SemaphoreType.DMA,
        ],
    )
    def k(in_hbm, idx_hbm, val_hbm, out_hbm, acc_sh, idx_v, val_v,
          stage_a, stage_b, sem_pair, sem_pull, sem_push):
        tid = lax.axis_index("s")
        stages = (stage_a, stage_b)

        pair0 = tid * PAIRS_PER_TILE
        cp_idx = pltpu.async_copy(idx_hbm.at[pl.ds(pair0, PAIRS_PER_TILE)],
                                  idx_v, sem_pair)
        cp_val = pltpu.async_copy(val_hbm.at[pl.ds(pair0, PAIRS_PER_TILE)],
                                  val_v, sem_pair)

        def staged_pipeline(src, dst, chunk):
            pulls = [None] * NSTAGE
            pushes = [None] * NSTAGE
            for c in range(NSTAGE):
                buf = stages[c % 2].at[pl.ds(0, chunk)]
                off = tid * SEG + c * chunk
                if c >= 2:
                    pushes[c - 2].wait()
                pulls[c] = pltpu.async_copy(src.at[pl.ds(off, chunk)], buf,
                                            sem_pull)
                pulls[c].wait()
                pushes[c] = pltpu.async_copy(buf, dst.at[pl.ds(off, chunk)],
                                             sem_push)
            pushes[NSTAGE - 2].wait()
            pushes[NSTAGE - 1].wait()

        @pl.when(tid < NT - 1)
        def _():
            staged_pipeline(in_hbm, acc_sh, STG)

        @pl.when(tid == NT - 1)
        def _():
            staged_pipeline(in_hbm, acc_sh, STG_LAST)

        cp_idx.wait()
        cp_val.wait()
        plsc.subcore_barrier()

        pltpu.sync_copy(val_v, acc_sh.at[idx_v], add=True)

        plsc.subcore_barrier()

        @pl.when(tid < NT - 1)
        def _():
            staged_pipeline(acc_sh, out_hbm, STG)

        @pl.when(tid == NT - 1)
        def _():
            staged_pipeline(acc_sh, out_hbm, STG_LAST)

    return k(inp, idx, val)


def kernel(input, index, value):
    return _sc_scatter_add(input, index, value)


# speedup vs baseline: 1.4527x; 1.0452x over previous
import functools

import jax
import jax.numpy as jnp
from jax import lax
from jax.experimental import pallas as pl
from jax.experimental.pallas import tpu as pltpu
from jax.experimental.pallas import tpu_sc as plsc

N = 1_000_000
NPAIR = 262_144

NT = 16
SEG = 62_528
NSTAGE = 4
STG = SEG // NSTAGE

PAIRS_PER_TILE = NPAIR // NT


def _sc_scatter_add(inp, idx, val):
    mesh = plsc.VectorSubcoreMesh(core_axis_name="c", subcore_axis_name="s",
                                  num_cores=1)

    @functools.partial(
        pl.kernel,
        mesh=mesh,
        out_type=jax.ShapeDtypeStruct((N,), jnp.int32),
        scratch_types=[
            pltpu.VMEM_SHARED((N,), jnp.int32),
            pltpu.VMEM((PAIRS_PER_TILE,), jnp.int32),
            pltpu.VMEM((PAIRS_PER_TILE,), jnp.int32),
            pltpu.VMEM((STG,), jnp.int32),
            pltpu.VMEM((STG,), jnp.int32),
            pltpu.SemaphoreType.DMA,
            pltpu.SemaphoreType.DMA,
            pltpu.SemaphoreType.DMA,
        ],
    )
    def k(in_hbm, idx_hbm, val_hbm, out_hbm, acc_sh, idx_v, val_v,
          stage_a, stage_b, sem_pair, sem_pull, sem_push):
        tid = lax.axis_index("s")
        bufs = (stage_a, stage_b)

        def off(c):
            o = tid * SEG + c * STG
            if c == NSTAGE - 1:
                o = jnp.minimum(o, N - STG)
            return o

        def pipeline(src, dst, first_extra=None):
            pulls = [None] * NSTAGE
            pushes = [None] * NSTAGE
            pulls[0] = pltpu.async_copy(src.at[pl.ds(off(0), STG)], bufs[0],
                                        sem_pull)
            if first_extra is not None:
                first_extra()
            pulls[1] = pltpu.async_copy(src.at[pl.ds(off(1), STG)], bufs[1],
                                        sem_pull)
            for c in range(NSTAGE):
                pulls[c].wait()
                pushes[c] = pltpu.async_copy(bufs[c % 2],
                                             dst.at[pl.ds(off(c), STG)],
                                             sem_push)
                if c + 2 < NSTAGE:
                    pushes[c].wait()
                    pulls[c + 2] = pltpu.async_copy(
                        src.at[pl.ds(off(c + 2), STG)], bufs[c % 2], sem_pull)
            pushes[NSTAGE - 2].wait()
            pushes[NSTAGE - 1].wait()

        cps = []

        def load_pairs():
            pair0 = tid * PAIRS_PER_TILE
            cps.append(pltpu.async_copy(
                idx_hbm.at[pl.ds(pair0, PAIRS_PER_TILE)], idx_v, sem_pair))
            cps.append(pltpu.async_copy(
                val_hbm.at[pl.ds(pair0, PAIRS_PER_TILE)], val_v, sem_pair))

        pipeline(in_hbm, acc_sh, first_extra=load_pairs)

        for cp in cps:
            cp.wait()
        plsc.subcore_barrier()

        pltpu.sync_copy(val_v, acc_sh.at[idx_v], add=True)

        plsc.subcore_barrier()

        pipeline(acc_sh, out_hbm)

    return k(inp, idx, val)


def kernel(input, index, value):
    return _sc_scatter_add(input, index, value)
